# baseline (speedup 1.0000x reference)
import functools

import jax
import jax.numpy as jnp
from jax import lax
from jax.experimental import pallas as pl
from jax.experimental.pallas import tpu as pltpu
from jax.experimental.pallas import tpu_sc as plsc

MAX_LENGTH = 256
OUT_DIM = 64
BATCH = 4096
SEQ = 200

NUM_CORES = 2
NUM_SUBCORES = 16
NUM_WORKERS = NUM_CORES * NUM_SUBCORES

LANES = 16
D8 = OUT_DIM // 8
BTILES = BATCH // 128
BQ = BTILES // (NUM_WORKERS // D8)
BW = BQ * 128
NPV = BW // LANES


def _make_kernel():
  mesh = plsc.VectorSubcoreMesh(core_axis_name="c", subcore_axis_name="s")

  @functools.partial(
      pl.kernel,
      mesh=mesh,
      out_type=jax.ShapeDtypeStruct((SEQ, D8, BTILES, 8, 128), jnp.float32),
      scratch_types=[
          pltpu.VMEM((MAX_LENGTH * OUT_DIM,), jnp.float32),
          pltpu.VMEM((BW,), jnp.int32),
          pltpu.VMEM((BW,), jnp.int32),
          pltpu.VMEM((BQ, 8, 128), jnp.float32),
          pltpu.VMEM((BQ, 8, 128), jnp.float32),
          pltpu.SemaphoreType.DMA,
          pltpu.SemaphoreType.DMA,
          pltpu.SemaphoreType.DMA,
          pltpu.SemaphoreType.DMA,
      ],
      compiler_params=pltpu.CompilerParams(use_tc_tiling_on_sc=False,
                                           needs_layout_passes=False),
  )
  def pe_kernel(tab_hbm, post_hbm, out_hbm, table_v, posb0, posb1, blk0,
                blk1, psem0, psem1, ssem0, ssem1):
    wid = lax.axis_index("s") * NUM_CORES + lax.axis_index("c")
    d8 = wid % D8
    btq = wid // D8
    b0 = btq * BW
    bt0 = btq * BQ
    dbase = d8 * 8

    pltpu.sync_copy(tab_hbm, table_v)

    posb = (posb0, posb1)
    blk = (blk0, blk1)
    psem = (psem0, psem1)
    ssem = (ssem0, ssem1)

    def pos_dma(s, p):
      return pltpu.make_async_copy(post_hbm.at[s, pl.ds(b0, BW)], posb[p],
                                   psem[p])

    def store_dma(s, p):
      return pltpu.make_async_copy(blk[p],
                                   out_hbm.at[s, d8, pl.ds(bt0, BQ)],
                                   ssem[p])

    def build(p):
      for g in range(NPV):
        pv = posb[p][pl.ds(g * LANES, LANES)]
        idx0 = pv * OUT_DIM + dbase
        for d1 in range(8):
          vals = plsc.load_gather(table_v, [idx0 + d1])
          blk[p][g // 8, d1, pl.ds((g % 8) * LANES, LANES)] = vals

    pos_dma(0, 0).start()
    pos_dma(1, 1).start()

    def body(j, carry):
      for par in (0, 1):
        s = 2 * j + par
        pos_dma(s, par).wait()

        @pl.when(j > 0)
        def _():
          store_dma(s - 2, par).wait()

        build(par)
        store_dma(s, par).start()

        @pl.when(j < SEQ // 2 - 1)
        def _():
          pos_dma(s + 2, par).start()
      return carry

    lax.fori_loop(0, SEQ // 2, body, 0)

    store_dma(SEQ - 2, 0).wait()
    store_dma(SEQ - 1, 1).wait()

  return pe_kernel


_PE_KERNEL = _make_kernel()


def kernel(positions, table):
  posT = positions.astype(jnp.int32).T
  tab_flat = table.reshape(MAX_LENGTH * OUT_DIM)
  out5 = _PE_KERNEL(tab_flat, posT)
  return out5.transpose(2, 4, 0, 1, 3).reshape(BATCH, SEQ, OUT_DIM)


# speedup vs baseline: 2.3700x; 2.3700x over previous
import functools

import jax
import jax.numpy as jnp
from jax import lax
from jax.experimental import pallas as pl
from jax.experimental.pallas import tpu as pltpu
from jax.experimental.pallas import tpu_sc as plsc

MAX_LENGTH = 256
OUT_DIM = 64
BATCH = 4096
SEQ = 200

NUM_CORES = 2
NUM_SUBCORES = 16
NUM_WORKERS = NUM_CORES * NUM_SUBCORES

LANES = 16
D8 = OUT_DIM // 8
BTILES = BATCH // 128
BQ = BTILES // (NUM_WORKERS // D8)
BW = BQ * 128
NPV = BW // LANES


def _make_kernel():
  mesh = plsc.VectorSubcoreMesh(core_axis_name="c", subcore_axis_name="s")

  @functools.partial(
      pl.kernel,
      mesh=mesh,
      out_type=jax.ShapeDtypeStruct((SEQ, D8, BTILES, 8, 128), jnp.float32),
      scratch_types=[
          pltpu.VMEM((MAX_LENGTH * OUT_DIM,), jnp.float32),
          [pltpu.VMEM((MAX_LENGTH,), jnp.float32) for _ in range(8)],
          pltpu.VMEM((BW,), jnp.int32),
          pltpu.VMEM((BW,), jnp.int32),
          pltpu.VMEM((BQ, 8, 128), jnp.float32),
          pltpu.VMEM((BQ, 8, 128), jnp.float32),
          pltpu.SemaphoreType.DMA,
          pltpu.SemaphoreType.DMA,
          pltpu.SemaphoreType.DMA,
          pltpu.SemaphoreType.DMA,
      ],
      compiler_params=pltpu.CompilerParams(use_tc_tiling_on_sc=False,
                                           needs_layout_passes=False),
  )
  def pe_kernel(tab_hbm, post_hbm, out_hbm, table_v, tabcols, posb0, posb1,
                blk0, blk1, psem0, psem1, ssem0, ssem1):
    wid = lax.axis_index("s") * NUM_CORES + lax.axis_index("c")
    d8 = wid % D8
    btq = wid // D8
    b0 = btq * BW
    bt0 = btq * BQ
    dbase = d8 * 8

    pltpu.sync_copy(tab_hbm, table_v)

    iota = lax.iota(jnp.int32, LANES)
    for t in range(MAX_LENGTH // LANES):
      cbase = (iota + t * LANES) * OUT_DIM + dbase
      for d1 in range(8):
        tabcols[d1][pl.ds(t * LANES, LANES)] = plsc.load_gather(
            table_v, [cbase + d1])

    posb = (posb0, posb1)
    blk = (blk0, blk1)
    psem = (psem0, psem1)
    ssem = (ssem0, ssem1)

    def pos_dma(s, p):
      return pltpu.make_async_copy(post_hbm.at[s, pl.ds(b0, BW)], posb[p],
                                   psem[p])

    def store_dma(s, p):
      return pltpu.make_async_copy(blk[p],
                                   out_hbm.at[s, d8, pl.ds(bt0, BQ)],
                                   ssem[p])

    def build(p):
      for g in range(NPV):
        pv = posb[p][pl.ds(g * LANES, LANES)]
        for d1 in range(8):
          vals = plsc.load_gather(tabcols[d1], [pv])
          blk[p][g // 8, d1, pl.ds((g % 8) * LANES, LANES)] = vals

    pos_dma(0, 0).start()
    pos_dma(1, 1).start()

    def body(j, carry):
      for par in (0, 1):
        s = 2 * j + par
        pos_dma(s, par).wait()

        @pl.when(j > 0)
        def _():
          store_dma(s - 2, par).wait()

        build(par)
        store_dma(s, par).start()

        @pl.when(j < SEQ // 2 - 1)
        def _():
          pos_dma(s + 2, par).start()
      return carry

    lax.fori_loop(0, SEQ // 2, body, 0)

    store_dma(SEQ - 2, 0).wait()
    store_dma(SEQ - 1, 1).wait()

  return pe_kernel


_PE_KERNEL = _make_kernel()


def kernel(positions, table):
  posT = positions.astype(jnp.int32).T
  tab_flat = table.reshape(MAX_LENGTH * OUT_DIM)
  out5 = _PE_KERNEL(tab_flat, posT)
  return out5.transpose(2, 4, 0, 1, 3).reshape(BATCH, SEQ, OUT_DIM)


# speedup vs baseline: 2.5789x; 1.0881x over previous
import functools

import jax
import jax.numpy as jnp
from jax import lax
from jax.experimental import pallas as pl
from jax.experimental.pallas import tpu as pltpu
from jax.experimental.pallas import tpu_sc as plsc

MAX_LENGTH = 256
OUT_DIM = 64
BATCH = 4096
SEQ = 200

NUM_CORES = 2
NUM_SUBCORES = 16
NUM_WORKERS = NUM_CORES * NUM_SUBCORES

LANES = 16
D8 = OUT_DIM // 8
BTILES = BATCH // 128
BQ = BTILES // (NUM_WORKERS // D8)
BW = BQ * 128
NPV = BW // LANES


def _make_kernel():
  mesh = plsc.VectorSubcoreMesh(core_axis_name="c", subcore_axis_name="s")

  @functools.partial(
      pl.kernel,
      mesh=mesh,
      out_type=jax.ShapeDtypeStruct((SEQ, D8, BTILES, 8, 128), jnp.float32),
      scratch_types=[
          pltpu.VMEM((MAX_LENGTH * OUT_DIM,), jnp.float32),
          [pltpu.VMEM((MAX_LENGTH * LANES,), jnp.float32) for _ in range(8)],
          pltpu.VMEM((BW,), jnp.int32),
          pltpu.VMEM((BW,), jnp.int32),
          pltpu.VMEM((BQ, 8, 128), jnp.float32),
          pltpu.VMEM((BQ, 8, 128), jnp.float32),
          pltpu.SemaphoreType.DMA,
          pltpu.SemaphoreType.DMA,
          pltpu.SemaphoreType.DMA,
          pltpu.SemaphoreType.DMA,
      ],
      compiler_params=pltpu.CompilerParams(use_tc_tiling_on_sc=False,
                                           needs_layout_passes=False),
  )
  def pe_kernel(tab_hbm, post_hbm, out_hbm, table_v, tabcols, posb0, posb1,
                blk0, blk1, psem0, psem1, ssem0, ssem1):
    wid = lax.axis_index("s") * NUM_CORES + lax.axis_index("c")
    d8 = wid % D8
    btq = wid // D8
    b0 = btq * BW
    bt0 = btq * BQ
    dbase = d8 * 8

    pltpu.sync_copy(tab_hbm, table_v)

    iota = lax.iota(jnp.int32, LANES)

    def rep_body(p, carry):
      for d1 in range(8):
        v = plsc.load_gather(
            table_v, [jnp.full((LANES,), p * OUT_DIM + dbase + d1, jnp.int32)])
        tabcols[d1][pl.ds(p * LANES, LANES)] = v
      return carry

    lax.fori_loop(0, MAX_LENGTH, rep_body, 0)

    posb = (posb0, posb1)
    blk = (blk0, blk1)
    psem = (psem0, psem1)
    ssem = (ssem0, ssem1)

    def pos_dma(s, p):
      return pltpu.make_async_copy(post_hbm.at[s, pl.ds(b0, BW)], posb[p],
                                   psem[p])

    def store_dma(s, p):
      return pltpu.make_async_copy(blk[p],
                                   out_hbm.at[s, d8, pl.ds(bt0, BQ)],
                                   ssem[p])

    def build(p):
      for g in range(NPV):
        pv = posb[p][pl.ds(g * LANES, LANES)] * LANES + iota
        for d1 in range(8):
          vals = plsc.load_gather(tabcols[d1], [pv])
          blk[p][g // 8, d1, pl.ds((g % 8) * LANES, LANES)] = vals

    pos_dma(0, 0).start()
    pos_dma(1, 1).start()

    def body(j, carry):
      for par in (0, 1):
        s = 2 * j + par
        pos_dma(s, par).wait()

        @pl.when(j > 0)
        def _():
          store_dma(s - 2, par).wait()

        build(par)
        store_dma(s, par).start()

        @pl.when(j < SEQ // 2 - 1)
        def _():
          pos_dma(s + 2, par).start()
      return carry

    lax.fori_loop(0, SEQ // 2, body, 0)

    store_dma(SEQ - 2, 0).wait()
    store_dma(SEQ - 1, 1).wait()

  return pe_kernel


_PE_KERNEL = _make_kernel()


def kernel(positions, table):
  posT = positions.astype(jnp.int32).T
  tab_flat = table.reshape(MAX_LENGTH * OUT_DIM)
  out5 = _PE_KERNEL(tab_flat, posT)
  return out5.transpose(2, 4, 0, 1, 3).reshape(BATCH, SEQ, OUT_DIM)


# speedup vs baseline: 3.8696x; 1.5005x over previous
import functools

import jax
import jax.numpy as jnp
from jax import lax
from jax.experimental import pallas as pl
from jax.experimental.pallas import tpu as pltpu
from jax.experimental.pallas import tpu_sc as plsc

MAX_LENGTH = 256
OUT_DIM = 64
BATCH = 4096
SEQ = 200

NUM_CORES = 2
NUM_SUBCORES = 16
NUM_WORKERS = NUM_CORES * NUM_SUBCORES

LANES = 16
D8 = OUT_DIM // 8
BTILES = BATCH // 128
BQ = BTILES // (NUM_WORKERS // D8)
BW = BQ * 128
NPV = BW // LANES


def _make_kernel():
  mesh = plsc.VectorSubcoreMesh(core_axis_name="c", subcore_axis_name="s")

  @functools.partial(
      pl.kernel,
      mesh=mesh,
      out_type=jax.ShapeDtypeStruct((SEQ, D8, BTILES, 8, 128), jnp.float32),
      scratch_types=[
          pltpu.VMEM((MAX_LENGTH * OUT_DIM,), jnp.float32),
          [pltpu.VMEM((MAX_LENGTH * LANES,), jnp.float32) for _ in range(8)],
          pltpu.VMEM((BW,), jnp.int32),
          pltpu.VMEM((BW,), jnp.int32),
          pltpu.VMEM((BQ, 8, 128), jnp.float32),
          pltpu.VMEM((BQ, 8, 128), jnp.float32),
          pltpu.SemaphoreType.DMA,
          pltpu.SemaphoreType.DMA,
          pltpu.SemaphoreType.DMA,
          pltpu.SemaphoreType.DMA,
      ],
      compiler_params=pltpu.CompilerParams(use_tc_tiling_on_sc=False,
                                           needs_layout_passes=False),
  )
  def pe_kernel(tab_hbm, post_hbm, out_hbm, table_v, tabcols, posb0, posb1,
                blk0, blk1, psem0, psem1, ssem0, ssem1):
    wid = lax.axis_index("s") * NUM_CORES + lax.axis_index("c")
    d8 = wid % D8
    btq = wid // D8
    b0 = btq * BW
    bt0 = btq * BQ
    dbase = d8 * 8

    pltpu.sync_copy(tab_hbm, table_v)

    iota = lax.iota(jnp.int32, LANES)

    def rep_body(p, carry):
      for d1 in range(8):
        v = plsc.load_gather(
            table_v, [jnp.full((LANES,), p * OUT_DIM + dbase + d1, jnp.int32)])
        tabcols[d1][pl.ds(p * LANES, LANES)] = v
      return carry

    lax.fori_loop(0, MAX_LENGTH, rep_body, 0)

    posb = (posb0, posb1)
    blk = (blk0, blk1)
    psem = (psem0, psem1)
    ssem = (ssem0, ssem1)

    def pos_dma(s, p):
      return pltpu.make_async_copy(post_hbm.at[s, pl.ds(b0, BW)], posb[p],
                                   psem[p])

    def store_dma(s, p):
      return pltpu.make_async_copy(blk[p],
                                   out_hbm.at[s, d8, pl.ds(bt0, BQ)],
                                   ssem[p])

    def build(p):
      for g2 in range(NPV // 2):
        g0 = 2 * g2
        g1 = 2 * g2 + 1
        pv0 = posb[p][pl.ds(g0 * LANES, LANES)] * LANES + iota
        pv1 = posb[p][pl.ds(g1 * LANES, LANES)] * LANES + iota
        vals0 = [plsc.load_gather(tabcols[d1], [pv0]) for d1 in range(8)]
        vals1 = [plsc.load_gather(tabcols[d1], [pv1]) for d1 in range(8)]
        for d1 in range(8):
          blk[p][g0 // 8, d1, pl.ds((g0 % 8) * LANES, LANES)] = vals0[d1]
        for d1 in range(8):
          blk[p][g1 // 8, d1, pl.ds((g1 % 8) * LANES, LANES)] = vals1[d1]

    pos_dma(0, 0).start()
    pos_dma(1, 1).start()

    def body(j, carry):
      for par in (0, 1):
        s = 2 * j + par
        pos_dma(s, par).wait()

        @pl.when(j > 0)
        def _():
          store_dma(s - 2, par).wait()

        build(par)
        store_dma(s, par).start()

        @pl.when(j < SEQ // 2 - 1)
        def _():
          pos_dma(s + 2, par).start()
      return carry

    lax.fori_loop(0, SEQ // 2, body, 0)

    store_dma(SEQ - 2, 0).wait()
    store_dma(SEQ - 1, 1).wait()

  return pe_kernel


_PE_KERNEL = _make_kernel()


def kernel(positions, table):
  posT = positions.astype(jnp.int32).T
  tab_flat = table.reshape(MAX_LENGTH * OUT_DIM)
  out5 = _PE_KERNEL(tab_flat, posT)
  return out5.transpose(2, 4, 0, 1, 3).reshape(BATCH, SEQ, OUT_DIM)


# speedup vs baseline: 4.7551x; 1.2288x over previous
import functools

import jax
import jax.numpy as jnp
from jax import lax
from jax.experimental import pallas as pl
from jax.experimental.pallas import tpu as pltpu
from jax.experimental.pallas import tpu_sc as plsc

MAX_LENGTH = 256
OUT_DIM = 64
BATCH = 4096
SEQ = 200

NUM_CORES = 2
NUM_SUBCORES = 16
NUM_WORKERS = NUM_CORES * NUM_SUBCORES

LANES = 16
D8 = OUT_DIM // 8
BTILES = BATCH // 128
BQ = BTILES // (NUM_WORKERS // D8)
BW = BQ * 128
NPV = BW // LANES


def _make_kernel():
  mesh = plsc.VectorSubcoreMesh(core_axis_name="c", subcore_axis_name="s")

  @functools.partial(
      pl.kernel,
      mesh=mesh,
      out_type=jax.ShapeDtypeStruct((SEQ, D8, BTILES, 8, 128), jnp.float32),
      scratch_types=[
          pltpu.VMEM((MAX_LENGTH * OUT_DIM,), jnp.float32),
          [pltpu.VMEM((MAX_LENGTH * LANES,), jnp.float32) for _ in range(8)],
          pltpu.VMEM((BW,), jnp.int32),
          pltpu.VMEM((BW,), jnp.int32),
          pltpu.VMEM((BQ, 8, 128), jnp.float32),
          pltpu.VMEM((BQ, 8, 128), jnp.float32),
          pltpu.SemaphoreType.DMA,
          pltpu.SemaphoreType.DMA,
          pltpu.SemaphoreType.DMA,
          pltpu.SemaphoreType.DMA,
      ],
      compiler_params=pltpu.CompilerParams(use_tc_tiling_on_sc=False,
                                           needs_layout_passes=False),
  )
  def pe_kernel(tab_hbm, post_hbm, out_hbm, table_v, tabcols, posb0, posb1,
                blk0, blk1, psem0, psem1, ssem0, ssem1):
    wid = lax.axis_index("s") * NUM_CORES + lax.axis_index("c")
    d8 = wid % D8
    btq = wid // D8
    b0 = btq * BW
    bt0 = btq * BQ
    dbase = d8 * 8

    pltpu.sync_copy(tab_hbm, table_v)

    iota = lax.iota(jnp.int32, LANES)

    def rep_body(p, carry):
      for d1 in range(8):
        v = plsc.load_gather(
            table_v, [jnp.full((LANES,), p * OUT_DIM + dbase + d1, jnp.int32)])
        tabcols[d1][pl.ds(p * LANES, LANES)] = v
      return carry

    lax.fori_loop(0, MAX_LENGTH, rep_body, 0)

    posb = (posb0, posb1)
    blk = (blk0, blk1)
    psem = (psem0, psem1)
    ssem = (ssem0, ssem1)

    def pos_dma(s, p):
      return pltpu.make_async_copy(post_hbm.at[s, pl.ds(b0, BW)], posb[p],
                                   psem[p])

    def store_dma(s, p):
      return pltpu.make_async_copy(blk[p],
                                   out_hbm.at[s, d8, pl.ds(bt0, BQ)],
                                   ssem[p])

    def build(p):
      for g2 in range(NPV // 2):
        g0 = 2 * g2
        g1 = 2 * g2 + 1
        pv0 = posb[p][pl.ds(g0 * LANES, LANES)] * LANES + iota
        pv1 = posb[p][pl.ds(g1 * LANES, LANES)] * LANES + iota
        vals0 = [plsc.load_gather(tabcols[d1], [pv0]) for d1 in range(8)]
        vals1 = []
        for d1 in range(8):
          vals1.append(plsc.load_gather(tabcols[d1], [pv1]))
          blk[p][g0 // 8, d1, pl.ds((g0 % 8) * LANES, LANES)] = vals0[d1]
        for d1 in range(8):
          blk[p][g1 // 8, d1, pl.ds((g1 % 8) * LANES, LANES)] = vals1[d1]

    pos_dma(0, 0).start()
    pos_dma(1, 1).start()

    def body(j, carry):
      for par in (0, 1):
        s = 2 * j + par
        pos_dma(s, par).wait()

        @pl.when(j > 0)
        def _():
          store_dma(s - 2, par).wait()

        build(par)
        store_dma(s, par).start()

        @pl.when(j < SEQ // 2 - 1)
        def _():
          pos_dma(s + 2, par).start()
      return carry

    lax.fori_loop(0, SEQ // 2, body, 0)

    store_dma(SEQ - 2, 0).wait()
    store_dma(SEQ - 1, 1).wait()

  return pe_kernel


_PE_KERNEL = _make_kernel()


def kernel(positions, table):
  posT = positions.astype(jnp.int32).T
  tab_flat = table.reshape(MAX_LENGTH * OUT_DIM)
  out5 = _PE_KERNEL(tab_flat, posT)
  return out5.transpose(2, 4, 0, 1, 3).reshape(BATCH, SEQ, OUT_DIM)


# speedup vs baseline: 7.6271x; 1.6040x over previous
import functools

import jax
import jax.numpy as jnp
from jax import lax
from jax.experimental import pallas as pl
from jax.experimental.pallas import tpu as pltpu
from jax.experimental.pallas import tpu_sc as plsc

MAX_LENGTH = 256
OUT_DIM = 64
BATCH = 4096
SEQ = 200

NUM_CORES = 2
NUM_SUBCORES = 16
NUM_WORKERS = NUM_CORES * NUM_SUBCORES

LANES = 16
D8 = OUT_DIM // 8
BTILES = BATCH // 128
BQ = BTILES // (NUM_WORKERS // D8)
BW = BQ * 128
NPV = BW // LANES


def _make_kernel():
  mesh = plsc.VectorSubcoreMesh(core_axis_name="c", subcore_axis_name="s")

  @functools.partial(
      pl.kernel,
      mesh=mesh,
      out_type=jax.ShapeDtypeStruct((SEQ, D8, BTILES, 8, 128), jnp.float32),
      scratch_types=[
          pltpu.VMEM((MAX_LENGTH * OUT_DIM,), jnp.float32),
          [pltpu.VMEM((MAX_LENGTH * LANES,), jnp.float32) for _ in range(8)],
          pltpu.VMEM((BW,), jnp.int32),
          pltpu.VMEM((BW,), jnp.int32),
          pltpu.VMEM((BQ, 8, 128), jnp.float32),
          pltpu.VMEM((BQ, 8, 128), jnp.float32),
          pltpu.SemaphoreType.DMA,
          pltpu.SemaphoreType.DMA,
          pltpu.SemaphoreType.DMA,
          pltpu.SemaphoreType.DMA,
      ],
      compiler_params=pltpu.CompilerParams(use_tc_tiling_on_sc=False,
                                           needs_layout_passes=False),
  )
  def pe_kernel(tab_hbm, post_hbm, out_hbm, table_v, tabcols, posb0, posb1,
                blk0, blk1, psem0, psem1, ssem0, ssem1):
    wid = lax.axis_index("s") * NUM_CORES + lax.axis_index("c")
    d8 = wid % D8
    btq = wid // D8
    b0 = btq * BW
    bt0 = btq * BQ
    dbase = d8 * 8

    pltpu.sync_copy(tab_hbm, table_v)

    iota = lax.iota(jnp.int32, LANES)

    def rep_body(p, carry):
      for d1 in range(8):
        v = plsc.load_gather(
            table_v, [jnp.full((LANES,), p * OUT_DIM + dbase + d1, jnp.int32)])
        tabcols[d1][pl.ds(p * LANES, LANES)] = v
      return carry

    lax.fori_loop(0, MAX_LENGTH, rep_body, 0)

    posb = (posb0, posb1)
    blk = (blk0, blk1)
    psem = (psem0, psem1)
    ssem = (ssem0, ssem1)

    def pos_dma(s, p):
      return pltpu.make_async_copy(post_hbm.at[s, pl.ds(b0, BW)], posb[p],
                                   psem[p])

    def store_dma(s, p):
      return pltpu.make_async_copy(blk[p],
                                   out_hbm.at[s, d8, pl.ds(bt0, BQ)],
                                   ssem[p])

    def build(p):
      pv = posb[p][pl.ds(0, LANES)] + iota
      vals = [plsc.load_gather(tabcols[d1], [pv]) for d1 in range(8)]
      for g in range(1, NPV):
        pv = posb[p][pl.ds(g * LANES, LANES)] + iota
        nvals = []
        for d1 in range(8):
          nvals.append(plsc.load_gather(tabcols[d1], [pv]))
          blk[p][(g - 1) // 8, d1,
                 pl.ds(((g - 1) % 8) * LANES, LANES)] = vals[d1]
        vals = nvals
      for d1 in range(8):
        blk[p][(NPV - 1) // 8, d1,
               pl.ds(((NPV - 1) % 8) * LANES, LANES)] = vals[d1]

    pos_dma(0, 0).start()
    pos_dma(1, 1).start()

    def body(j, carry):
      for par in (0, 1):
        s = 2 * j + par
        pos_dma(s, par).wait()

        @pl.when(j > 0)
        def _():
          store_dma(s - 2, par).wait()

        build(par)
        store_dma(s, par).start()

        @pl.when(j < SEQ // 2 - 1)
        def _():
          pos_dma(s + 2, par).start()
      return carry

    lax.fori_loop(0, SEQ // 2, body, 0)

    store_dma(SEQ - 2, 0).wait()
    store_dma(SEQ - 1, 1).wait()

  return pe_kernel


_PE_KERNEL = _make_kernel()


def kernel(positions, table):
  posT = positions.astype(jnp.int32).T * LANES
  tab_flat = table.reshape(MAX_LENGTH * OUT_DIM)
  out5 = _PE_KERNEL(tab_flat, posT)
  return out5.transpose(2, 4, 0, 1, 3).reshape(BATCH, SEQ, OUT_DIM)


# speedup vs baseline: 7.7995x; 1.0226x over previous
import functools

import jax
import jax.numpy as jnp
from jax import lax
from jax.experimental import pallas as pl
from jax.experimental.pallas import tpu as pltpu
from jax.experimental.pallas import tpu_sc as plsc

MAX_LENGTH = 256
OUT_DIM = 64
BATCH = 4096
SEQ = 200

NUM_CORES = 2
NUM_SUBCORES = 16
NUM_WORKERS = NUM_CORES * NUM_SUBCORES

LANES = 16
D8 = OUT_DIM // 8
BTILES = BATCH // 128
BQ = BTILES // (NUM_WORKERS // D8)
BW = BQ * 128
NPV = BW // LANES


def _make_kernel():
  mesh = plsc.VectorSubcoreMesh(core_axis_name="c", subcore_axis_name="s")

  @functools.partial(
      pl.kernel,
      mesh=mesh,
      out_type=jax.ShapeDtypeStruct((SEQ, D8, BTILES, 8, 128), jnp.float32),
      scratch_types=[
          [pltpu.VMEM((MAX_LENGTH * LANES,), jnp.float32) for _ in range(8)],
          pltpu.VMEM((BW,), jnp.int32),
          pltpu.VMEM((BW,), jnp.int32),
          pltpu.VMEM((BQ, 8, 128), jnp.float32),
          pltpu.VMEM((BQ, 8, 128), jnp.float32),
          pltpu.SemaphoreType.DMA,
          pltpu.SemaphoreType.DMA,
          pltpu.SemaphoreType.DMA,
          pltpu.SemaphoreType.DMA,
      ],
      compiler_params=pltpu.CompilerParams(use_tc_tiling_on_sc=False,
                                           needs_layout_passes=False),
  )
  def pe_kernel(tab_hbm, post_hbm, out_hbm, tabcols, posb0, posb1,
                blk0, blk1, psem0, psem1, ssem0, ssem1):
    wid = lax.axis_index("s") * NUM_CORES + lax.axis_index("c")
    d8 = wid % D8
    btq = wid // D8
    b0 = btq * BW
    bt0 = btq * BQ
    dbase = d8 * 8

    iota = lax.iota(jnp.int32, LANES)
    for d1 in range(8):
      pltpu.sync_copy(tab_hbm.at[dbase + d1], tabcols[d1])

    posb = (posb0, posb1)
    blk = (blk0, blk1)
    psem = (psem0, psem1)
    ssem = (ssem0, ssem1)

    def pos_dma(s, p):
      return pltpu.make_async_copy(post_hbm.at[s, pl.ds(b0, BW)], posb[p],
                                   psem[p])

    def store_dma(s, p):
      return pltpu.make_async_copy(blk[p],
                                   out_hbm.at[s, d8, pl.ds(bt0, BQ)],
                                   ssem[p])

    def build(p):
      pv = posb[p][pl.ds(0, LANES)] + iota
      vals = [plsc.load_gather(tabcols[d1], [pv]) for d1 in range(8)]
      for g in range(1, NPV):
        pv = posb[p][pl.ds(g * LANES, LANES)] + iota
        nvals = []
        for d1 in range(8):
          nvals.append(plsc.load_gather(tabcols[d1], [pv]))
          blk[p][(g - 1) // 8, d1,
                 pl.ds(((g - 1) % 8) * LANES, LANES)] = vals[d1]
        vals = nvals
      for d1 in range(8):
        blk[p][(NPV - 1) // 8, d1,
               pl.ds(((NPV - 1) % 8) * LANES, LANES)] = vals[d1]

    pos_dma(0, 0).start()
    pos_dma(1, 1).start()

    def body(j, carry):
      for par in (0, 1):
        s = 2 * j + par
        pos_dma(s, par).wait()

        @pl.when(j > 0)
        def _():
          store_dma(s - 2, par).wait()

        build(par)
        store_dma(s, par).start()

        @pl.when(j < SEQ // 2 - 1)
        def _():
          pos_dma(s + 2, par).start()
      return carry

    lax.fori_loop(0, SEQ // 2, body, 0)

    store_dma(SEQ - 2, 0).wait()
    store_dma(SEQ - 1, 1).wait()

  return pe_kernel


_PE_KERNEL = _make_kernel()


def kernel(positions, table):
  posT = positions.astype(jnp.int32).T * LANES
  tabrep = jnp.broadcast_to(table.T[:, :, None],
                            (OUT_DIM, MAX_LENGTH, LANES))
  tabrep = tabrep.reshape(OUT_DIM, MAX_LENGTH * LANES)
  out5 = _PE_KERNEL(tabrep, posT)
  return out5.transpose(2, 4, 0, 1, 3).reshape(BATCH, SEQ, OUT_DIM)
